# baseline (speedup 1.0000x reference)
import functools

import jax
import jax.numpy as jnp
from jax import lax
from jax.experimental import pallas as pl
from jax.experimental.pallas import tpu as pltpu
from jax.experimental.pallas import tpu_sc as plsc

N = 1_000_000
L = 16
NC = 2
NS = 16
NW = NC * NS
NCH = 6
CV = 328
CVE = CV * L
VPW = CV * NCH
CPW = VPW * L
UNROLL = 8

_DNUMS = lax.GatherDimensionNumbers(
    offset_dims=(), collapsed_slice_dims=(0,), start_index_map=(0,)
)


def _gather16(table, idx):
    return lax.gather(
        table,
        idx[:, None],
        _DNUMS,
        slice_sizes=(1,),
        mode=lax.GatherScatterMode.PROMISE_IN_BOUNDS,
    )


def _make_kernel():
    mesh = plsc.VectorSubcoreMesh(core_axis_name="c", subcore_axis_name="s")

    @functools.partial(
        pl.kernel,
        mesh=mesh,
        out_type=jax.ShapeDtypeStruct((N,), jnp.float32),
        scratch_types=[
            pltpu.VMEM((CPW,), jnp.float32),
            pltpu.VMEM((CPW,), jnp.int32),
            pltpu.VMEM((CPW,), jnp.float32),
            pltpu.VMEM((L,), jnp.float32),
            pltpu.VMEM((L,), jnp.float32),
            pltpu.SemaphoreType.DMA((NCH,)),
            pltpu.SemaphoreType.DMA,
            pltpu.SemaphoreType.DMA,
        ],
    )
    def k(x_hbm, sp_hbm, scale_hbm, shift_hbm, out_hbm, x_v, sp_v, o_v, tscale, tshift, in_sems, tab_sem, out_sem):
        wid = lax.axis_index("s") * NC + lax.axis_index("c")
        base = jnp.minimum(wid * CPW, N - CPW)

        pltpu.async_copy(scale_hbm, tscale, tab_sem)
        pltpu.async_copy(shift_hbm, tshift, tab_sem)

        def enqueue(g, c):
            sl = pl.ds(base + g * CVE, CVE)
            vl = pl.ds(g * CVE, CVE)
            pltpu.async_copy(x_hbm.at[sl], x_v.at[vl], in_sems.at[g])
            pltpu.async_copy(sp_hbm.at[sl], sp_v.at[vl], in_sems.at[g])
            return c

        lax.fori_loop(0, NCH, enqueue, 0)

        pltpu.make_async_copy(scale_hbm, tscale, tab_sem).wait()
        pltpu.make_async_copy(shift_hbm, tshift, tab_sem).wait()
        scale_vec = tscale[...]
        shift_vec = tshift[...]

        def chunk(g, c):
            sl = pl.ds(base + g * CVE, CVE)
            vl = pl.ds(g * CVE, CVE)
            pltpu.make_async_copy(x_hbm.at[sl], x_v.at[vl], in_sems.at[g]).wait()
            pltpu.make_async_copy(sp_hbm.at[sl], sp_v.at[vl], in_sems.at[g]).wait()

            def inner(j, ci):
                for u in range(UNROLL):
                    vsl = pl.ds((g * CV + j * UNROLL + u) * L, L)
                    idx = sp_v[vsl]
                    xs = x_v[vsl]
                    o_v[vsl] = (
                        _gather16(scale_vec, idx) * xs + _gather16(shift_vec, idx)
                    )
                return ci

            lax.fori_loop(0, CV // UNROLL, inner, 0)
            pltpu.async_copy(o_v.at[vl], out_hbm.at[sl], out_sem)
            return c

        lax.fori_loop(0, NCH, chunk, 0)

        def drain(g, c):
            sl = pl.ds(base + g * CVE, CVE)
            vl = pl.ds(g * CVE, CVE)
            pltpu.make_async_copy(o_v.at[vl], out_hbm.at[sl], out_sem).wait()
            return c

        lax.fori_loop(0, NCH, drain, 0)

    return k


_scale_shift = _make_kernel()


def kernel(x, species, scale_params, shift_params):
    return _scale_shift(x, species, scale_params, shift_params)


# speedup vs baseline: 1.1623x; 1.1623x over previous
import functools

import jax
import jax.numpy as jnp
from jax import lax
from jax.experimental import pallas as pl
from jax.experimental.pallas import tpu as pltpu
from jax.experimental.pallas import tpu_sc as plsc

N = 1_000_000
L = 16
NC = 2
NS = 16
NW = NC * NS
NCH = 4
CV = 492
CVE = CV * L
VPW = CV * NCH
CPW = VPW * L
UNROLL = 6

_DNUMS = lax.GatherDimensionNumbers(
    offset_dims=(), collapsed_slice_dims=(0,), start_index_map=(0,)
)


def _gather16(table, idx):
    return lax.gather(
        table,
        idx[:, None],
        _DNUMS,
        slice_sizes=(1,),
        mode=lax.GatherScatterMode.PROMISE_IN_BOUNDS,
    )


def _make_kernel():
    mesh = plsc.VectorSubcoreMesh(core_axis_name="c", subcore_axis_name="s")

    @functools.partial(
        pl.kernel,
        mesh=mesh,
        out_type=jax.ShapeDtypeStruct((N,), jnp.float32),
        scratch_types=[
            pltpu.VMEM((CPW,), jnp.float32),
            pltpu.VMEM((CPW,), jnp.int32),
            pltpu.VMEM((CPW,), jnp.float32),
            pltpu.VMEM((L,), jnp.float32),
            pltpu.VMEM((L,), jnp.float32),
        ]
        + [pltpu.SemaphoreType.DMA] * (NCH + 2),
    )
    def k(x_hbm, sp_hbm, scale_hbm, shift_hbm, out_hbm, x_v, sp_v, o_v, tscale, tshift, *sems):
        in_sems = sems[:NCH]
        tab_sem = sems[NCH]
        out_sem = sems[NCH + 1]
        wid = lax.axis_index("s") * NC + lax.axis_index("c")
        base = jnp.minimum(wid * CPW, N - CPW)

        pltpu.async_copy(scale_hbm, tscale, tab_sem)
        pltpu.async_copy(shift_hbm, tshift, tab_sem)
        for g in range(NCH):
            sl = pl.ds(base + g * CVE, CVE)
            vl = pl.ds(g * CVE, CVE)
            pltpu.async_copy(x_hbm.at[sl], x_v.at[vl], in_sems[g])
            pltpu.async_copy(sp_hbm.at[sl], sp_v.at[vl], in_sems[g])

        pltpu.make_async_copy(scale_hbm, tscale, tab_sem).wait()
        pltpu.make_async_copy(shift_hbm, tshift, tab_sem).wait()
        scale_vec = tscale[...]
        shift_vec = tshift[...]

        for g in range(NCH):
            sl = pl.ds(base + g * CVE, CVE)
            vl = pl.ds(g * CVE, CVE)
            pltpu.make_async_copy(x_hbm.at[sl], x_v.at[vl], in_sems[g]).wait()
            pltpu.make_async_copy(sp_hbm.at[sl], sp_v.at[vl], in_sems[g]).wait()

            def inner(j, c, g=g):
                for u in range(UNROLL):
                    vsl = pl.ds((g * CV + j * UNROLL + u) * L, L)
                    idx = sp_v[vsl]
                    xs = x_v[vsl]
                    o_v[vsl] = (
                        _gather16(scale_vec, idx) * xs + _gather16(shift_vec, idx)
                    )
                return c

            lax.fori_loop(0, CV // UNROLL, inner, 0)
            pltpu.async_copy(o_v.at[vl], out_hbm.at[sl], out_sem)

        for g in range(NCH):
            sl = pl.ds(base + g * CVE, CVE)
            vl = pl.ds(g * CVE, CVE)
            pltpu.make_async_copy(o_v.at[vl], out_hbm.at[sl], out_sem).wait()

    return k


_scale_shift = _make_kernel()


def kernel(x, species, scale_params, shift_params):
    return _scale_shift(x, species, scale_params, shift_params)


# speedup vs baseline: 1.4924x; 1.2840x over previous
import functools

import jax
import jax.numpy as jnp
from jax import lax
from jax.experimental import pallas as pl
from jax.experimental.pallas import tpu as pltpu
from jax.experimental.pallas import tpu_sc as plsc

N = 1_000_000
L = 16
NC = 2
NS = 16
NW = NC * NS
NCH = 8
CV = 248
CVE = CV * L
VPW = CV * NCH
CPW = VPW * L
UNROLL = 8

_DNUMS = lax.GatherDimensionNumbers(
    offset_dims=(), collapsed_slice_dims=(0,), start_index_map=(0,)
)


def _gather16(table, idx):
    return lax.gather(
        table,
        idx[:, None],
        _DNUMS,
        slice_sizes=(1,),
        mode=lax.GatherScatterMode.PROMISE_IN_BOUNDS,
    )


def _make_kernel():
    mesh = plsc.VectorSubcoreMesh(core_axis_name="c", subcore_axis_name="s")

    @functools.partial(
        pl.kernel,
        mesh=mesh,
        out_type=jax.ShapeDtypeStruct((N,), jnp.float32),
        scratch_types=[
            pltpu.VMEM((CPW,), jnp.float32),
            pltpu.VMEM((CPW,), jnp.int32),
            pltpu.VMEM((CPW,), jnp.float32),
            pltpu.VMEM((L,), jnp.float32),
            pltpu.VMEM((L,), jnp.float32),
        ]
        + [pltpu.SemaphoreType.DMA] * (NCH + 2),
    )
    def k(x_hbm, sp_hbm, scale_hbm, shift_hbm, out_hbm, x_v, sp_v, o_v, tscale, tshift, *sems):
        in_sems = sems[:NCH]
        tab_sem = sems[NCH]
        out_sem = sems[NCH + 1]
        wid = lax.axis_index("s") * NC + lax.axis_index("c")
        base = jnp.minimum(wid * CPW, N - CPW)

        pltpu.async_copy(scale_hbm, tscale, tab_sem)
        pltpu.async_copy(shift_hbm, tshift, tab_sem)
        for g in range(NCH):
            sl = pl.ds(base + g * CVE, CVE)
            vl = pl.ds(g * CVE, CVE)
            pltpu.async_copy(x_hbm.at[sl], x_v.at[vl], in_sems[g])
            pltpu.async_copy(sp_hbm.at[sl], sp_v.at[vl], in_sems[g])

        pltpu.make_async_copy(scale_hbm, tscale, tab_sem).wait()
        pltpu.make_async_copy(shift_hbm, tshift, tab_sem).wait()
        scale_vec = tscale[...]
        shift_vec = tshift[...]

        for g in range(NCH):
            sl = pl.ds(base + g * CVE, CVE)
            vl = pl.ds(g * CVE, CVE)
            pltpu.make_async_copy(x_hbm.at[sl], x_v.at[vl], in_sems[g]).wait()
            pltpu.make_async_copy(sp_hbm.at[sl], sp_v.at[vl], in_sems[g]).wait()

            def inner(j, c, g=g):
                for u in range(UNROLL):
                    vsl = pl.ds((g * CV + j * UNROLL + u) * L, L)
                    idx = sp_v[vsl]
                    xs = x_v[vsl]
                    o_v[vsl] = (
                        _gather16(scale_vec, idx) * xs + _gather16(shift_vec, idx)
                    )
                return c

            lax.fori_loop(0, CV // UNROLL, inner, 0)
            pltpu.async_copy(o_v.at[vl], out_hbm.at[sl], out_sem)

        for g in range(NCH):
            sl = pl.ds(base + g * CVE, CVE)
            vl = pl.ds(g * CVE, CVE)
            pltpu.make_async_copy(o_v.at[vl], out_hbm.at[sl], out_sem).wait()

    return k


_scale_shift = _make_kernel()


def kernel(x, species, scale_params, shift_params):
    return _scale_shift(x, species, scale_params, shift_params)
